# baseline (speedup 1.0000x reference)
import jax
import jax.numpy as jnp
from jax.experimental import pallas as pl
from jax.experimental.pallas import tpu as pltpu

N_NODES_C = 100000
N_GRAPHS_C = 256
D_C = 128
BLOCK_R = 1000
N_BLOCKS = N_NODES_C // BLOCK_R


def _gnavg_kernel(ids_ref, x_ref, u_ref, w1a_ref, w1b_ref, b1_ref, w2_ref,
                  b2_ref, out_ref, sums_ref, cnts_ref):
    i = pl.program_id(0)

    @pl.when(i == 0)
    def _():
        sums_ref[...] = jnp.zeros_like(sums_ref)
        cnts_ref[...] = jnp.zeros_like(cnts_ref)

    ids = ids_ref[0, 0, :]
    seg = jax.lax.broadcasted_iota(jnp.int32, (BLOCK_R, N_GRAPHS_C), 1)
    onehot = (ids[:, None] == seg).astype(jnp.bfloat16)
    xb = x_ref[...]
    hi = xb.astype(jnp.bfloat16)
    lo = (xb - hi.astype(jnp.float32)).astype(jnp.bfloat16)
    psum = jax.lax.dot_general(
        onehot, hi, (((0,), (0,)), ((), ())),
        preferred_element_type=jnp.float32)
    psum += jax.lax.dot_general(
        onehot, lo, (((0,), (0,)), ((), ())),
        preferred_element_type=jnp.float32)
    sums_ref[...] += psum
    ones = jnp.ones((BLOCK_R, 8), jnp.bfloat16)
    pcnt = jax.lax.dot_general(
        onehot, ones, (((0,), (0,)), ((), ())),
        preferred_element_type=jnp.float32)
    cnts_ref[...] += pcnt

    @pl.when(i == N_BLOCKS - 1)
    def _():
        counts = cnts_ref[:, 0:1]
        mean = sums_ref[...] / jnp.maximum(counts, 1.0)
        u1 = jax.lax.dot_general(
            mean, w1a_ref[...], (((1,), (0,)), ((), ())),
            preferred_element_type=jnp.float32)
        u1 += jax.lax.dot_general(
            u_ref[...], w1b_ref[...], (((1,), (0,)), ((), ())),
            preferred_element_type=jnp.float32)
        u1 = jnp.maximum(u1 + b1_ref[...], 0.0)
        out = jax.lax.dot_general(
            u1, w2_ref[...], (((1,), (0,)), ((), ())),
            preferred_element_type=jnp.float32)
        out_ref[...] = out + b2_ref[...]


@jax.jit
def kernel(x, u, graph_ids, W1, b1, W2, b2):
    ids = graph_ids.astype(jnp.int32).reshape(N_BLOCKS, 1, BLOCK_R)
    w1a = W1[:D_C]
    w1b = W1[D_C:]
    b1r = b1.reshape(1, D_C)
    b2r = b2.reshape(1, D_C)
    rep = lambda shape: pl.BlockSpec(shape, lambda i: (0,) * len(shape))
    return pl.pallas_call(
        _gnavg_kernel,
        grid=(N_BLOCKS,),
        in_specs=[
            pl.BlockSpec((1, 1, BLOCK_R), lambda i: (i, 0, 0)),
            pl.BlockSpec((BLOCK_R, D_C), lambda i: (i, 0)),
            rep((N_GRAPHS_C, D_C)),
            rep((D_C, D_C)),
            rep((D_C, D_C)),
            rep((1, D_C)),
            rep((D_C, D_C)),
            rep((1, D_C)),
        ],
        out_specs=rep((N_GRAPHS_C, D_C)),
        out_shape=jax.ShapeDtypeStruct((N_GRAPHS_C, D_C), jnp.float32),
        scratch_shapes=[
            pltpu.VMEM((N_GRAPHS_C, D_C), jnp.float32),
            pltpu.VMEM((N_GRAPHS_C, 8), jnp.float32),
        ],
    )(ids, x, u, w1a, w1b, b1r, W2, b2r)


# speedup vs baseline: 1.1556x; 1.1556x over previous
import jax
import jax.numpy as jnp
from jax.experimental import pallas as pl
from jax.experimental.pallas import tpu as pltpu

N_NODES_C = 100000
N_GRAPHS_C = 256
D_C = 128
BLOCK_R = 1000
N_BLOCKS = N_NODES_C // BLOCK_R


def _gnavg_kernel(ids_ref, x_ref, u_ref, w1a_ref, w1b_ref, b1_ref, w2_ref,
                  b2_ref, out_ref, sums_ref, cnts_ref):
    i = pl.program_id(0)

    @pl.when(i == 0)
    def _():
        sums_ref[...] = jnp.zeros_like(sums_ref)
        cnts_ref[...] = jnp.zeros_like(cnts_ref)

    ids = ids_ref[0, :, :]
    seg = jax.lax.broadcasted_iota(jnp.int32, (N_GRAPHS_C, BLOCK_R), 0)
    onehot = (ids == seg).astype(jnp.bfloat16)
    xb = x_ref[...]
    hi = xb.astype(jnp.bfloat16)
    lo = (xb - hi.astype(jnp.float32)).astype(jnp.bfloat16)
    psum = jax.lax.dot_general(
        onehot, hi, (((1,), (0,)), ((), ())),
        preferred_element_type=jnp.float32)
    psum += jax.lax.dot_general(
        onehot, lo, (((1,), (0,)), ((), ())),
        preferred_element_type=jnp.float32)
    sums_ref[...] += psum
    ones = jnp.ones((BLOCK_R, 8), jnp.bfloat16)
    pcnt = jax.lax.dot_general(
        onehot, ones, (((1,), (0,)), ((), ())),
        preferred_element_type=jnp.float32)
    cnts_ref[...] += pcnt

    @pl.when(i == N_BLOCKS - 1)
    def _():
        counts = cnts_ref[:, 0:1]
        mean = sums_ref[...] / jnp.maximum(counts, 1.0)
        u1 = jax.lax.dot_general(
            mean, w1a_ref[...], (((1,), (0,)), ((), ())),
            preferred_element_type=jnp.float32)
        u1 += jax.lax.dot_general(
            u_ref[...], w1b_ref[...], (((1,), (0,)), ((), ())),
            preferred_element_type=jnp.float32)
        u1 = jnp.maximum(u1 + b1_ref[...], 0.0)
        out = jax.lax.dot_general(
            u1, w2_ref[...], (((1,), (0,)), ((), ())),
            preferred_element_type=jnp.float32)
        out_ref[...] = out + b2_ref[...]


@jax.jit
def kernel(x, u, graph_ids, W1, b1, W2, b2):
    ids = graph_ids.astype(jnp.int32).reshape(N_BLOCKS, 1, BLOCK_R)
    w1a = W1[:D_C]
    w1b = W1[D_C:]
    b1r = b1.reshape(1, D_C)
    b2r = b2.reshape(1, D_C)
    rep = lambda shape: pl.BlockSpec(shape, lambda i: (0,) * len(shape))
    return pl.pallas_call(
        _gnavg_kernel,
        grid=(N_BLOCKS,),
        in_specs=[
            pl.BlockSpec((1, 1, BLOCK_R), lambda i: (i, 0, 0)),
            pl.BlockSpec((BLOCK_R, D_C), lambda i: (i, 0)),
            rep((N_GRAPHS_C, D_C)),
            rep((D_C, D_C)),
            rep((D_C, D_C)),
            rep((1, D_C)),
            rep((D_C, D_C)),
            rep((1, D_C)),
        ],
        out_specs=rep((N_GRAPHS_C, D_C)),
        out_shape=jax.ShapeDtypeStruct((N_GRAPHS_C, D_C), jnp.float32),
        scratch_shapes=[
            pltpu.VMEM((N_GRAPHS_C, D_C), jnp.float32),
            pltpu.VMEM((N_GRAPHS_C, 8), jnp.float32),
        ],
    )(ids, x, u, w1a, w1b, b1r, W2, b2r)


# speedup vs baseline: 1.8874x; 1.6332x over previous
import jax
import jax.numpy as jnp
from jax.experimental import pallas as pl
from jax.experimental.pallas import tpu as pltpu

N_NODES_C = 100000
N_GRAPHS_C = 256
D_C = 128
BLOCK_R = 2000
N_BLOCKS = N_NODES_C // BLOCK_R


def _gnavg_kernel(ids_ref, x_ref, u_ref, w1a_ref, w1b_ref, b1_ref, w2_ref,
                  b2_ref, out_ref, sums_ref, cnts_ref):
    i = pl.program_id(0)

    @pl.when(i == 0)
    def _():
        sums_ref[...] = jnp.zeros_like(sums_ref)
        cnts_ref[...] = jnp.zeros_like(cnts_ref)

    ids = ids_ref[0, :, :].astype(jnp.int16)
    seg = jax.lax.broadcasted_iota(jnp.int16, (N_GRAPHS_C, BLOCK_R), 0)
    onehot = jnp.where(ids == seg, jnp.bfloat16(1.0), jnp.bfloat16(0.0))
    xb = x_ref[...].astype(jnp.bfloat16)
    psum = jax.lax.dot_general(
        onehot, xb, (((1,), (0,)), ((), ())),
        preferred_element_type=jnp.float32)
    sums_ref[...] += psum
    ones = jnp.ones((BLOCK_R, 8), jnp.bfloat16)
    pcnt = jax.lax.dot_general(
        onehot, ones, (((1,), (0,)), ((), ())),
        preferred_element_type=jnp.float32)
    cnts_ref[...] += pcnt

    @pl.when(i == N_BLOCKS - 1)
    def _():
        counts = cnts_ref[:, 0:1]
        mean = sums_ref[...] / jnp.maximum(counts, 1.0)
        u1 = jax.lax.dot_general(
            mean, w1a_ref[...], (((1,), (0,)), ((), ())),
            preferred_element_type=jnp.float32)
        u1 += jax.lax.dot_general(
            u_ref[...], w1b_ref[...], (((1,), (0,)), ((), ())),
            preferred_element_type=jnp.float32)
        u1 = jnp.maximum(u1 + b1_ref[...], 0.0)
        out = jax.lax.dot_general(
            u1, w2_ref[...], (((1,), (0,)), ((), ())),
            preferred_element_type=jnp.float32)
        out_ref[...] = out + b2_ref[...]


@jax.jit
def kernel(x, u, graph_ids, W1, b1, W2, b2):
    ids = graph_ids.astype(jnp.int32).reshape(N_BLOCKS, 1, BLOCK_R)
    w1a = W1[:D_C]
    w1b = W1[D_C:]
    b1r = b1.reshape(1, D_C)
    b2r = b2.reshape(1, D_C)
    rep = lambda shape: pl.BlockSpec(shape, lambda i: (0,) * len(shape))
    return pl.pallas_call(
        _gnavg_kernel,
        grid=(N_BLOCKS,),
        in_specs=[
            pl.BlockSpec((1, 1, BLOCK_R), lambda i: (i, 0, 0)),
            pl.BlockSpec((BLOCK_R, D_C), lambda i: (i, 0)),
            rep((N_GRAPHS_C, D_C)),
            rep((D_C, D_C)),
            rep((D_C, D_C)),
            rep((1, D_C)),
            rep((D_C, D_C)),
            rep((1, D_C)),
        ],
        out_specs=rep((N_GRAPHS_C, D_C)),
        out_shape=jax.ShapeDtypeStruct((N_GRAPHS_C, D_C), jnp.float32),
        scratch_shapes=[
            pltpu.VMEM((N_GRAPHS_C, D_C), jnp.float32),
            pltpu.VMEM((N_GRAPHS_C, 8), jnp.float32),
        ],
    )(ids, x, u, w1a, w1b, b1r, W2, b2r)


# speedup vs baseline: 2.6347x; 1.3960x over previous
import jax
import jax.numpy as jnp
from jax.experimental import pallas as pl
from jax.experimental.pallas import tpu as pltpu

N_NODES_C = 100000
N_GRAPHS_C = 256
D_C = 128
BLOCK_R = 5000
N_BLOCKS = N_NODES_C // BLOCK_R


def _gnavg_kernel(ids_ref, x_ref, u_ref, w1a_ref, w1b_ref, b1_ref, w2_ref,
                  b2_ref, out_ref, sums_ref, cnts_ref):
    i = pl.program_id(0)

    @pl.when(i == 0)
    def _():
        sums_ref[...] = jnp.zeros_like(sums_ref)
        cnts_ref[...] = jnp.zeros_like(cnts_ref)

    ids = ids_ref[0, :, :].astype(jnp.int16)
    seg = jax.lax.broadcasted_iota(jnp.int16, (N_GRAPHS_C, BLOCK_R), 0)
    onehot = jnp.where(ids == seg, jnp.bfloat16(1.0), jnp.bfloat16(0.0))
    xb = x_ref[...].astype(jnp.bfloat16)
    psum = jax.lax.dot_general(
        onehot, xb, (((1,), (0,)), ((), ())),
        preferred_element_type=jnp.float32)
    sums_ref[...] += psum
    ones = jnp.ones((BLOCK_R, 8), jnp.bfloat16)
    pcnt = jax.lax.dot_general(
        onehot, ones, (((1,), (0,)), ((), ())),
        preferred_element_type=jnp.float32)
    cnts_ref[...] += pcnt

    @pl.when(i == N_BLOCKS - 1)
    def _():
        counts = cnts_ref[:, 0:1]
        mean = sums_ref[...] / jnp.maximum(counts, 1.0)
        u1 = jax.lax.dot_general(
            mean, w1a_ref[...], (((1,), (0,)), ((), ())),
            preferred_element_type=jnp.float32)
        u1 += jax.lax.dot_general(
            u_ref[...], w1b_ref[...], (((1,), (0,)), ((), ())),
            preferred_element_type=jnp.float32)
        u1 = jnp.maximum(u1 + b1_ref[...], 0.0)
        out = jax.lax.dot_general(
            u1, w2_ref[...], (((1,), (0,)), ((), ())),
            preferred_element_type=jnp.float32)
        out_ref[...] = out + b2_ref[...]


@jax.jit
def kernel(x, u, graph_ids, W1, b1, W2, b2):
    ids = graph_ids.astype(jnp.int32).reshape(N_BLOCKS, 1, BLOCK_R)
    w1a = W1[:D_C]
    w1b = W1[D_C:]
    b1r = b1.reshape(1, D_C)
    b2r = b2.reshape(1, D_C)
    rep = lambda shape: pl.BlockSpec(shape, lambda i: (0,) * len(shape))
    return pl.pallas_call(
        _gnavg_kernel,
        grid=(N_BLOCKS,),
        in_specs=[
            pl.BlockSpec((1, 1, BLOCK_R), lambda i: (i, 0, 0)),
            pl.BlockSpec((BLOCK_R, D_C), lambda i: (i, 0)),
            rep((N_GRAPHS_C, D_C)),
            rep((D_C, D_C)),
            rep((D_C, D_C)),
            rep((1, D_C)),
            rep((D_C, D_C)),
            rep((1, D_C)),
        ],
        out_specs=rep((N_GRAPHS_C, D_C)),
        out_shape=jax.ShapeDtypeStruct((N_GRAPHS_C, D_C), jnp.float32),
        scratch_shapes=[
            pltpu.VMEM((N_GRAPHS_C, D_C), jnp.float32),
            pltpu.VMEM((N_GRAPHS_C, 8), jnp.float32),
        ],
    )(ids, x, u, w1a, w1b, b1r, W2, b2r)


# speedup vs baseline: 2.8539x; 1.0832x over previous
import jax
import jax.numpy as jnp
from jax.experimental import pallas as pl
from jax.experimental.pallas import tpu as pltpu

N_NODES_C = 100000
N_GRAPHS_C = 256
D_C = 128
BLOCK_R = 10000
N_BLOCKS = N_NODES_C // BLOCK_R


def _gnavg_kernel(ids_ref, x_ref, u_ref, w1a_ref, w1b_ref, b1_ref, w2_ref,
                  b2_ref, out_ref, sums_ref, cnts_ref):
    i = pl.program_id(0)

    @pl.when(i == 0)
    def _():
        sums_ref[...] = jnp.zeros_like(sums_ref)
        cnts_ref[...] = jnp.zeros_like(cnts_ref)

    ids = ids_ref[0, :, :].astype(jnp.int16)
    seg = jax.lax.broadcasted_iota(jnp.int16, (N_GRAPHS_C, BLOCK_R), 0)
    onehot = jnp.where(ids == seg, jnp.bfloat16(1.0), jnp.bfloat16(0.0))
    xb = x_ref[...].astype(jnp.bfloat16)
    psum = jax.lax.dot_general(
        onehot, xb, (((1,), (0,)), ((), ())),
        preferred_element_type=jnp.float32)
    sums_ref[...] += psum
    ones = jnp.ones((BLOCK_R, 8), jnp.bfloat16)
    pcnt = jax.lax.dot_general(
        onehot, ones, (((1,), (0,)), ((), ())),
        preferred_element_type=jnp.float32)
    cnts_ref[...] += pcnt

    @pl.when(i == N_BLOCKS - 1)
    def _():
        counts = cnts_ref[:, 0:1]
        mean = sums_ref[...] / jnp.maximum(counts, 1.0)
        u1 = jax.lax.dot_general(
            mean, w1a_ref[...], (((1,), (0,)), ((), ())),
            preferred_element_type=jnp.float32)
        u1 += jax.lax.dot_general(
            u_ref[...], w1b_ref[...], (((1,), (0,)), ((), ())),
            preferred_element_type=jnp.float32)
        u1 = jnp.maximum(u1 + b1_ref[...], 0.0)
        out = jax.lax.dot_general(
            u1, w2_ref[...], (((1,), (0,)), ((), ())),
            preferred_element_type=jnp.float32)
        out_ref[...] = out + b2_ref[...]


@jax.jit
def kernel(x, u, graph_ids, W1, b1, W2, b2):
    ids = graph_ids.astype(jnp.int32).reshape(N_BLOCKS, 1, BLOCK_R)
    w1a = W1[:D_C]
    w1b = W1[D_C:]
    b1r = b1.reshape(1, D_C)
    b2r = b2.reshape(1, D_C)
    rep = lambda shape: pl.BlockSpec(shape, lambda i: (0,) * len(shape))
    return pl.pallas_call(
        _gnavg_kernel,
        grid=(N_BLOCKS,),
        in_specs=[
            pl.BlockSpec((1, 1, BLOCK_R), lambda i: (i, 0, 0)),
            pl.BlockSpec((BLOCK_R, D_C), lambda i: (i, 0)),
            rep((N_GRAPHS_C, D_C)),
            rep((D_C, D_C)),
            rep((D_C, D_C)),
            rep((1, D_C)),
            rep((D_C, D_C)),
            rep((1, D_C)),
        ],
        out_specs=rep((N_GRAPHS_C, D_C)),
        out_shape=jax.ShapeDtypeStruct((N_GRAPHS_C, D_C), jnp.float32),
        scratch_shapes=[
            pltpu.VMEM((N_GRAPHS_C, D_C), jnp.float32),
            pltpu.VMEM((N_GRAPHS_C, 8), jnp.float32),
        ],
    )(ids, x, u, w1a, w1b, b1r, W2, b2r)
